# Initial kernel scaffold; baseline (speedup 1.0000x reference)
import jax
import jax.numpy as jnp
from jax.experimental import pallas as pl


def kernel(source, target, src_labels, tar_labels, weigth, src_centroid):
    raise NotImplementedError("write your pallas kernel here")


# speedup vs baseline: 4.3127x; 4.3127x over previous
import functools

import jax
import jax.numpy as jnp
from jax import lax
from jax.experimental import pallas as pl
from jax.experimental.pallas import tpu as pltpu
from jax.experimental.pallas import tpu_sc as plsc

C = 1000
D = 128
B = 16384
DECAY = 0.9
TEMP = 1e-06

NC = 2
NS = 16
NW = NC * NS
CHUNK = B // NW
NG = CHUNK // 128
ROWS_BUF = 256


def _sc_body(src_r, tar_r, sl_r, tl_r, zcd, ones_hbm,
             sum_s_out, sum_t_out, cnt_s_out, cnt_t_out,
             rows_v, idx_s_v, idx_t_v, ones_v,
             acc_s, acc_t, hcnt_s, hcnt_t):
    c = lax.axis_index("c")
    s = lax.axis_index("s")
    wid = s * NC + c

    pltpu.sync_copy(sl_r.at[wid], idx_s_v)
    pltpu.sync_copy(tl_r.at[wid], idx_t_v)
    pltpu.sync_copy(ones_hbm, ones_v)

    @pl.when(s == 0)
    def _():
        pltpu.sync_copy(zcd, acc_s)
        pltpu.sync_copy(zcd, acc_t)
        pltpu.sync_copy(zcd, hcnt_s)
        pltpu.sync_copy(zcd, hcnt_t)

    plsc.subcore_barrier()

    for h in range(CHUNK // ROWS_BUF):
        pltpu.sync_copy(src_r.at[wid, h], rows_v)
        for j in range(ROWS_BUF // 128):
            g = h * (ROWS_BUF // 128) + j
            pltpu.sync_copy(rows_v.at[pl.ds(j * 128, 128)],
                            acc_s.at[idx_s_v.at[g]], add=True)
            pltpu.sync_copy(ones_v, hcnt_s.at[idx_s_v.at[g]], add=True)
    for h in range(CHUNK // ROWS_BUF):
        pltpu.sync_copy(tar_r.at[wid, h], rows_v)
        for j in range(ROWS_BUF // 128):
            g = h * (ROWS_BUF // 128) + j
            pltpu.sync_copy(rows_v.at[pl.ds(j * 128, 128)],
                            acc_t.at[idx_t_v.at[g]], add=True)
            pltpu.sync_copy(ones_v, hcnt_t.at[idx_t_v.at[g]], add=True)

    plsc.subcore_barrier()

    @pl.when(s == 0)
    def _():
        pltpu.sync_copy(acc_s, sum_s_out.at[c])
        pltpu.sync_copy(acc_t, sum_t_out.at[c])
        pltpu.sync_copy(hcnt_s, cnt_s_out.at[c])
        pltpu.sync_copy(hcnt_t, cnt_t_out.at[c])


@functools.lru_cache(maxsize=1)
def _get_sc_call():
    return functools.partial(
        pl.kernel,
        mesh=plsc.VectorSubcoreMesh(core_axis_name="c", subcore_axis_name="s"),
        out_type=[
            jax.ShapeDtypeStruct((NC, C, D), jnp.float32),
            jax.ShapeDtypeStruct((NC, C, D), jnp.float32),
            jax.ShapeDtypeStruct((NC, C, D), jnp.float32),
            jax.ShapeDtypeStruct((NC, C, D), jnp.float32),
        ],
        scratch_types=[
            pltpu.VMEM((ROWS_BUF, D), jnp.float32),
            pltpu.VMEM((NG, 128), jnp.int32),
            pltpu.VMEM((NG, 128), jnp.int32),
            pltpu.VMEM((128, D), jnp.float32),
            pltpu.VMEM_SHARED((C, D), jnp.float32),
            pltpu.VMEM_SHARED((C, D), jnp.float32),
            pltpu.VMEM_SHARED((C, D), jnp.float32),
            pltpu.VMEM_SHARED((C, D), jnp.float32),
        ],
    )(_sc_body)


def _tc_body(ssum, tsum, csrc, ctar, pcol, prow, w_ref, cent_ref, out_ref):
    neg_inf = jnp.float32(-jnp.inf)
    sum_src = ssum[0] + ssum[1]
    sum_tar = tsum[0] + tsum[1]
    cs = jnp.maximum(csrc[...], 1.0)
    ct = jnp.maximum(ctar[...], 1.0)
    mean_src = sum_src / cs
    mean_tar = sum_tar / ct
    cent = cent_ref[...]

    def nrm(x):
        n = jnp.sqrt(jnp.sum(x * x, axis=1, keepdims=True))
        return x / jnp.maximum(n, 1e-12)

    final_src = DECAY * cent + (1.0 - DECAY) * mean_src
    final_tar = (1.0 - DECAY) * nrm(cent) + DECAY * nrm(mean_tar)
    s_mat = nrm(final_src)
    t_mat = nrm(final_tar)
    sim = lax.dot_general(s_mat, t_mat, (((1,), (1,)), ((), ())),
                          preferred_element_type=jnp.float32)

    pm = jnp.logical_and(pcol[...], prow[...])
    w = jnp.where(pm, w_ref[...], neg_inf)
    m0 = jnp.max(w, axis=0, keepdims=True)
    e0 = jnp.exp(w - m0)
    w0 = e0 / jnp.sum(e0, axis=0, keepdims=True) + TEMP
    m1 = jnp.max(w, axis=1, keepdims=True)
    e1 = jnp.exp(w - m1)
    w1 = e1 / jnp.sum(e1, axis=1, keepdims=True) + TEMP

    sim2 = (sim * w0 + sim * w1) * 0.5
    sim2 = jnp.where(pm, sim2, neg_inf)

    mm = jnp.max(sim2, axis=1, keepdims=True)
    lse = jnp.log(jnp.sum(jnp.exp(sim2 - mm), axis=1, keepdims=True)) + mm
    rows_i = lax.broadcasted_iota(jnp.int32, (C, C), 0)
    cols_i = lax.broadcasted_iota(jnp.int32, (C, C), 1)
    eye = rows_i == cols_i
    diag_sim = jnp.sum(jnp.where(eye, sim2, 0.0), axis=1, keepdims=True)
    diag_logp = diag_sim - lse

    pf = pcol[...].astype(jnp.float32)
    k = jnp.sum(pf)
    loss = -jnp.sum(jnp.where(pcol[...], diag_logp, 0.0)) / k
    out_ref[...] = jnp.broadcast_to(loss, (8, 128))


_tc_call = pl.pallas_call(
    _tc_body,
    out_shape=jax.ShapeDtypeStruct((8, 128), jnp.float32),
)


@jax.jit
def kernel(source, target, src_labels, tar_labels, weigth, src_centroid):
    src_r = source.reshape(NW, CHUNK // ROWS_BUF, ROWS_BUF, D)
    tar_r = target.reshape(NW, CHUNK // ROWS_BUF, ROWS_BUF, D)
    sl_r = src_labels.reshape(NW, NG, 128)
    tl_r = tar_labels.reshape(NW, NG, 128)
    zcd = jnp.zeros((C, D), jnp.float32)
    ones = jnp.ones((128, D), jnp.float32)

    sum_s_p, sum_t_p, cnt_s_p, cnt_t_p = _get_sc_call()(
        src_r, tar_r, sl_r, tl_r, zcd, ones)

    cnt_src = cnt_s_p[0, :, 0] + cnt_s_p[1, :, 0]
    cnt_tar = cnt_t_p[0, :, 0] + cnt_t_p[1, :, 0]
    present = (cnt_src > 0) & (cnt_tar > 0)
    pcol = present.reshape(C, 1)
    prow = present.reshape(1, C)

    loss = _tc_call(sum_s_p, sum_t_p,
                    cnt_src.reshape(C, 1), cnt_tar.reshape(C, 1),
                    pcol, prow, weigth, src_centroid)
    return loss[0, 0]


# speedup vs baseline: 5.9677x; 1.3838x over previous
import functools

import jax
import jax.numpy as jnp
from jax import lax
from jax.experimental import pallas as pl
from jax.experimental.pallas import tpu as pltpu
from jax.experimental.pallas import tpu_sc as plsc

C = 1000
D = 128
B = 16384
DECAY = 0.9
TEMP = 1e-06

NC = 2
NS = 16
NW = NC * NS
CHUNK = B // NW
NG = CHUNK // 128
NBUF = 4
CP = 1008


def _sc_body(src_r, tar_r, sl_r, tl_r, zcd,
             sum_s_out, sum_t_out, hist_s_out, hist_t_out,
             buf0, buf1, buf2, buf3, idx_s_v, idx_t_v, hist_s_v, hist_t_v,
             acc_s, acc_t,
             sem_idx, sem_r0, sem_r1, sem_r2, sem_r3,
             sem_c0, sem_c1, sem_c2, sem_c3):
    c = lax.axis_index("c")
    s = lax.axis_index("s")
    wid = s * NC + c
    bufs = [buf0, buf1, buf2, buf3]
    sem_rows = [sem_r0, sem_r1, sem_r2, sem_r3]
    sem_scat = [sem_c0, sem_c1, sem_c2, sem_c3]

    cp_is = pltpu.async_copy(sl_r.at[wid], idx_s_v, sem_idx)
    cp_it = pltpu.async_copy(tl_r.at[wid], idx_t_v, sem_idx)
    row_cp = {}
    for g in range(NBUF):
        row_cp[g] = pltpu.async_copy(src_r.at[wid, g], bufs[g], sem_rows[g])

    z16 = jnp.zeros((16,), jnp.float32)
    for i in range(CP // 16):
        hist_s_v[pl.ds(i * 16, 16)] = z16
        hist_t_v[pl.ds(i * 16, 16)] = z16

    @pl.when(s == 0)
    def _():
        pltpu.sync_copy(zcd, acc_s)
        pltpu.sync_copy(zcd, acc_t)

    cp_is.wait()
    cp_it.wait()
    plsc.subcore_barrier()

    ones16 = jnp.ones((16,), jnp.float32)
    scat_cp = {}
    for g in range(2 * NG):
        b = g % NBUF
        row_cp[g].wait()
        if g < NG:
            idx_v, acc, hist_v = idx_s_v, acc_s, hist_s_v
            grp = g
        else:
            idx_v, acc, hist_v = idx_t_v, acc_t, hist_t_v
            grp = g - NG
        scat_cp[g] = pltpu.async_copy(bufs[b], acc.at[idx_v.at[grp]],
                                      sem_scat[b], add=True)
        for k in range(8):
            lab = idx_v[grp, pl.ds(k * 16, 16)]
            plsc.addupdate_scatter(hist_v, [lab], ones16)
        nxt = g + NBUF
        if nxt < 2 * NG:
            scat_cp[g].wait()
            if nxt < NG:
                row_cp[nxt] = pltpu.async_copy(src_r.at[wid, nxt], bufs[b],
                                               sem_rows[b])
            else:
                row_cp[nxt] = pltpu.async_copy(tar_r.at[wid, nxt - NG],
                                               bufs[b], sem_rows[b])
    for g in range(2 * NG - NBUF, 2 * NG):
        scat_cp[g].wait()

    pltpu.sync_copy(hist_s_v, hist_s_out.at[c, s])
    pltpu.sync_copy(hist_t_v, hist_t_out.at[c, s])

    plsc.subcore_barrier()

    @pl.when(s == 0)
    def _():
        pltpu.sync_copy(acc_s, sum_s_out.at[c])
        pltpu.sync_copy(acc_t, sum_t_out.at[c])


@functools.lru_cache(maxsize=1)
def _get_sc_call():
    return functools.partial(
        pl.kernel,
        mesh=plsc.VectorSubcoreMesh(core_axis_name="c", subcore_axis_name="s"),
        out_type=[
            jax.ShapeDtypeStruct((NC, C, D), jnp.float32),
            jax.ShapeDtypeStruct((NC, C, D), jnp.float32),
            jax.ShapeDtypeStruct((NC, NS, CP), jnp.float32),
            jax.ShapeDtypeStruct((NC, NS, CP), jnp.float32),
        ],
        scratch_types=[
            pltpu.VMEM((128, D), jnp.float32),
            pltpu.VMEM((128, D), jnp.float32),
            pltpu.VMEM((128, D), jnp.float32),
            pltpu.VMEM((128, D), jnp.float32),
            pltpu.VMEM((NG, 128), jnp.int32),
            pltpu.VMEM((NG, 128), jnp.int32),
            pltpu.VMEM((CP,), jnp.float32),
            pltpu.VMEM((CP,), jnp.float32),
            pltpu.VMEM_SHARED((C, D), jnp.float32),
            pltpu.VMEM_SHARED((C, D), jnp.float32),
            pltpu.SemaphoreType.DMA,
            pltpu.SemaphoreType.DMA,
            pltpu.SemaphoreType.DMA,
            pltpu.SemaphoreType.DMA,
            pltpu.SemaphoreType.DMA,
            pltpu.SemaphoreType.DMA,
            pltpu.SemaphoreType.DMA,
            pltpu.SemaphoreType.DMA,
            pltpu.SemaphoreType.DMA,
        ],
        compiler_params=pltpu.CompilerParams(needs_layout_passes=False),
    )(_sc_body)


def _tc_body(ssum, tsum, csrc, ctar, pcol, prow, w_ref, cent_ref, out_ref):
    neg_inf = jnp.float32(-jnp.inf)
    sum_src = ssum[0] + ssum[1]
    sum_tar = tsum[0] + tsum[1]
    cs = jnp.maximum(csrc[...], 1.0)
    ct = jnp.maximum(ctar[...], 1.0)
    mean_src = sum_src / cs
    mean_tar = sum_tar / ct
    cent = cent_ref[...]

    def nrm(x):
        n = jnp.sqrt(jnp.sum(x * x, axis=1, keepdims=True))
        return x / jnp.maximum(n, 1e-12)

    final_src = DECAY * cent + (1.0 - DECAY) * mean_src
    final_tar = (1.0 - DECAY) * nrm(cent) + DECAY * nrm(mean_tar)
    s_mat = nrm(final_src)
    t_mat = nrm(final_tar)
    sim = lax.dot_general(s_mat, t_mat, (((1,), (1,)), ((), ())),
                          preferred_element_type=jnp.float32)

    pm = jnp.logical_and(pcol[...], prow[...])
    w = jnp.where(pm, w_ref[...], neg_inf)
    m0 = jnp.max(w, axis=0, keepdims=True)
    e0 = jnp.exp(w - m0)
    w0 = e0 / jnp.sum(e0, axis=0, keepdims=True) + TEMP
    m1 = jnp.max(w, axis=1, keepdims=True)
    e1 = jnp.exp(w - m1)
    w1 = e1 / jnp.sum(e1, axis=1, keepdims=True) + TEMP

    sim2 = (sim * w0 + sim * w1) * 0.5
    sim2 = jnp.where(pm, sim2, neg_inf)

    mm = jnp.max(sim2, axis=1, keepdims=True)
    lse = jnp.log(jnp.sum(jnp.exp(sim2 - mm), axis=1, keepdims=True)) + mm
    rows_i = lax.broadcasted_iota(jnp.int32, (C, C), 0)
    cols_i = lax.broadcasted_iota(jnp.int32, (C, C), 1)
    eye = rows_i == cols_i
    diag_sim = jnp.sum(jnp.where(eye, sim2, 0.0), axis=1, keepdims=True)
    diag_logp = diag_sim - lse

    pf = pcol[...].astype(jnp.float32)
    k = jnp.sum(pf)
    loss = -jnp.sum(jnp.where(pcol[...], diag_logp, 0.0)) / k
    out_ref[...] = jnp.broadcast_to(loss, (8, 128))


_tc_call = pl.pallas_call(
    _tc_body,
    out_shape=jax.ShapeDtypeStruct((8, 128), jnp.float32),
)


@jax.jit
def kernel(source, target, src_labels, tar_labels, weigth, src_centroid):
    src_r = source.reshape(NW, NG, 128, D)
    tar_r = target.reshape(NW, NG, 128, D)
    sl_r = src_labels.reshape(NW, NG, 128)
    tl_r = tar_labels.reshape(NW, NG, 128)
    zcd = jnp.zeros((C, D), jnp.float32)

    sum_s_p, sum_t_p, hist_s, hist_t = _get_sc_call()(
        src_r, tar_r, sl_r, tl_r, zcd)

    cnt_src = jnp.sum(hist_s, axis=(0, 1))[:C]
    cnt_tar = jnp.sum(hist_t, axis=(0, 1))[:C]
    present = (cnt_src > 0) & (cnt_tar > 0)
    pcol = present.reshape(C, 1)
    prow = present.reshape(1, C)

    loss = _tc_call(sum_s_p, sum_t_p,
                    cnt_src.reshape(C, 1), cnt_tar.reshape(C, 1),
                    pcol, prow, weigth, src_centroid)
    return loss[0, 0]


# speedup vs baseline: 6.9309x; 1.1614x over previous
import functools

import jax
import jax.numpy as jnp
from jax import lax
from jax.experimental import pallas as pl
from jax.experimental.pallas import tpu as pltpu
from jax.experimental.pallas import tpu_sc as plsc

C = 1000
D = 128
B = 16384
DECAY = 0.9
TEMP = 1e-06

NC = 2
NS = 16
NW = NC * NS
CHUNK = B // NW
NG = CHUNK // 128
NBUF = 4
CP = 1008


def _sc_body(src_r, tar_r, sl_r, tl_r, zcd,
             sum_s_out, sum_t_out, hist_s_out, hist_t_out,
             buf0, buf1, buf2, buf3, idx_s_v, idx_t_v, hist_s_v, hist_t_v,
             acc_s, acc_t,
             sem_idx, sem_r0, sem_r1, sem_r2, sem_r3,
             sem_c0, sem_c1, sem_c2, sem_c3):
    c = lax.axis_index("c")
    s = lax.axis_index("s")
    wid = s * NC + c
    bufs = [buf0, buf1, buf2, buf3]
    sem_rows = [sem_r0, sem_r1, sem_r2, sem_r3]
    sem_scat = [sem_c0, sem_c1, sem_c2, sem_c3]

    cp_is = pltpu.async_copy(sl_r.at[wid], idx_s_v, sem_idx)
    cp_it = pltpu.async_copy(tl_r.at[wid], idx_t_v, sem_idx)
    row_cp = {}
    for g in range(NBUF):
        row_cp[g] = pltpu.async_copy(src_r.at[wid, g], bufs[g], sem_rows[g])

    z16 = jnp.zeros((16,), jnp.float32)
    for i in range(CP // 16):
        hist_s_v[pl.ds(i * 16, 16)] = z16
        hist_t_v[pl.ds(i * 16, 16)] = z16

    @pl.when(s == 0)
    def _():
        pltpu.sync_copy(zcd, acc_s)
        pltpu.sync_copy(zcd, acc_t)

    cp_is.wait()
    cp_it.wait()
    plsc.subcore_barrier()

    ones16 = jnp.ones((16,), jnp.float32)
    scat_cp = {}
    for g in range(2 * NG):
        b = g % NBUF
        row_cp[g].wait()
        if g < NG:
            idx_v, acc, hist_v = idx_s_v, acc_s, hist_s_v
            grp = g
        else:
            idx_v, acc, hist_v = idx_t_v, acc_t, hist_t_v
            grp = g - NG
        scat_cp[g] = pltpu.async_copy(bufs[b], acc.at[idx_v.at[grp]],
                                      sem_scat[b], add=True)
        for k in range(8):
            lab = idx_v[grp, pl.ds(k * 16, 16)]
            plsc.addupdate_scatter(hist_v, [lab], ones16)
        nxt = g + NBUF
        if nxt < 2 * NG:
            scat_cp[g].wait()
            if nxt < NG:
                row_cp[nxt] = pltpu.async_copy(src_r.at[wid, nxt], bufs[b],
                                               sem_rows[b])
            else:
                row_cp[nxt] = pltpu.async_copy(tar_r.at[wid, nxt - NG],
                                               bufs[b], sem_rows[b])
    for g in range(2 * NG - NBUF, 2 * NG):
        scat_cp[g].wait()

    pltpu.sync_copy(hist_s_v, hist_s_out.at[c, s])
    pltpu.sync_copy(hist_t_v, hist_t_out.at[c, s])

    plsc.subcore_barrier()

    @pl.when(s == 0)
    def _():
        pltpu.sync_copy(acc_s, sum_s_out.at[c])
        pltpu.sync_copy(acc_t, sum_t_out.at[c])


@functools.lru_cache(maxsize=1)
def _get_sc_call():
    return functools.partial(
        pl.kernel,
        mesh=plsc.VectorSubcoreMesh(core_axis_name="c", subcore_axis_name="s"),
        out_type=[
            jax.ShapeDtypeStruct((NC, C, D), jnp.float32),
            jax.ShapeDtypeStruct((NC, C, D), jnp.float32),
            jax.ShapeDtypeStruct((NC, NS, CP), jnp.float32),
            jax.ShapeDtypeStruct((NC, NS, CP), jnp.float32),
        ],
        scratch_types=[
            pltpu.VMEM((128, D), jnp.float32),
            pltpu.VMEM((128, D), jnp.float32),
            pltpu.VMEM((128, D), jnp.float32),
            pltpu.VMEM((128, D), jnp.float32),
            pltpu.VMEM((NG, 128), jnp.int32),
            pltpu.VMEM((NG, 128), jnp.int32),
            pltpu.VMEM((CP,), jnp.float32),
            pltpu.VMEM((CP,), jnp.float32),
            pltpu.VMEM_SHARED((C, D), jnp.float32),
            pltpu.VMEM_SHARED((C, D), jnp.float32),
            pltpu.SemaphoreType.DMA,
            pltpu.SemaphoreType.DMA,
            pltpu.SemaphoreType.DMA,
            pltpu.SemaphoreType.DMA,
            pltpu.SemaphoreType.DMA,
            pltpu.SemaphoreType.DMA,
            pltpu.SemaphoreType.DMA,
            pltpu.SemaphoreType.DMA,
            pltpu.SemaphoreType.DMA,
        ],
        compiler_params=pltpu.CompilerParams(needs_layout_passes=False),
    )(_sc_body)


def _tc_body(ssum, tsum, hs_ref, ht_ref, w_ref, cent_ref, out_ref):
    neg_inf = jnp.float32(-jnp.inf)
    sum_src = ssum[0] + ssum[1]
    sum_tar = tsum[0] + tsum[1]

    hs = hs_ref[...]
    ht = ht_ref[...]
    cnt_row_s = jnp.sum(hs, axis=0, keepdims=True)
    cnt_row_t = jnp.sum(ht, axis=0, keepdims=True)
    hsT = lax.transpose(hs, (1, 0))
    htT = lax.transpose(ht, (1, 0))
    csrc = jnp.sum(hsT, axis=1, keepdims=True)[:C]
    ctar = jnp.sum(htT, axis=1, keepdims=True)[:C]
    pcol = (csrc > 0) & (ctar > 0)
    prow = (cnt_row_s[:, :C] > 0) & (cnt_row_t[:, :C] > 0)

    cs = jnp.maximum(csrc, 1.0)
    ct = jnp.maximum(ctar, 1.0)
    mean_src = sum_src / cs
    mean_tar = sum_tar / ct
    cent = cent_ref[...]

    def nrm(x):
        n = jnp.sqrt(jnp.sum(x * x, axis=1, keepdims=True))
        return x / jnp.maximum(n, 1e-12)

    final_src = DECAY * cent + (1.0 - DECAY) * mean_src
    final_tar = (1.0 - DECAY) * nrm(cent) + DECAY * nrm(mean_tar)
    s_mat = nrm(final_src)
    t_mat = nrm(final_tar)
    sim = lax.dot_general(s_mat, t_mat, (((1,), (1,)), ((), ())),
                          preferred_element_type=jnp.float32)

    pm = jnp.logical_and(pcol, prow)
    w = jnp.where(pm, w_ref[...], neg_inf)
    m0 = jnp.max(w, axis=0, keepdims=True)
    e0 = jnp.exp(w - m0)
    w0 = e0 / jnp.sum(e0, axis=0, keepdims=True) + TEMP
    m1 = jnp.max(w, axis=1, keepdims=True)
    e1 = jnp.exp(w - m1)
    w1 = e1 / jnp.sum(e1, axis=1, keepdims=True) + TEMP

    sim2 = (sim * w0 + sim * w1) * 0.5
    sim2 = jnp.where(pm, sim2, neg_inf)

    mm = jnp.max(sim2, axis=1, keepdims=True)
    lse = jnp.log(jnp.sum(jnp.exp(sim2 - mm), axis=1, keepdims=True)) + mm
    rows_i = lax.broadcasted_iota(jnp.int32, (C, C), 0)
    cols_i = lax.broadcasted_iota(jnp.int32, (C, C), 1)
    eye = rows_i == cols_i
    diag_sim = jnp.sum(jnp.where(eye, sim2, 0.0), axis=1, keepdims=True)
    diag_logp = diag_sim - lse

    pf = pcol.astype(jnp.float32)
    k = jnp.sum(pf)
    loss = -jnp.sum(jnp.where(pcol, diag_logp, 0.0)) / k
    out_ref[0, 0] = loss


_tc_call = pl.pallas_call(
    _tc_body,
    out_shape=jax.ShapeDtypeStruct((1, 1), jnp.float32),
    out_specs=pl.BlockSpec(memory_space=pltpu.MemorySpace.SMEM),
)


@jax.jit
def kernel(source, target, src_labels, tar_labels, weigth, src_centroid):
    src_r = source.reshape(NW, NG, 128, D)
    tar_r = target.reshape(NW, NG, 128, D)
    sl_r = src_labels.reshape(NW, NG, 128)
    tl_r = tar_labels.reshape(NW, NG, 128)
    zcd = jnp.zeros((C, D), jnp.float32)

    sum_s_p, sum_t_p, hist_s, hist_t = _get_sc_call()(
        src_r, tar_r, sl_r, tl_r, zcd)

    loss = _tc_call(sum_s_p, sum_t_p,
                    hist_s.reshape(NW, CP), hist_t.reshape(NW, CP),
                    weigth, src_centroid)
    return loss[0, 0]
